# baseline (speedup 1.0000x reference)
import jax
import jax.numpy as jnp
from jax import lax
from jax.experimental import pallas as pl
from jax.experimental.pallas import tpu as pltpu
from jax.experimental.pallas import tpu_sc as plsc

N, E, CIN, COUT, T = 10000, 160000, 128, 16, 16
NP = 10240
HALF = T * COUT // 2

NC, NS = 2, 16
NW = NC * NS
C = 128
NCHUNK = 40
E_PER_W = NCHUNK * C
EP = NW * E_PER_W
EA_RPC = C * T // 128
ZROWS = 128

ROWS = 1024
GRID = NP // ROWS


def _tc_body(x_ref, g_ref, b_ref, m2a_ref, m2b_ref, m2c_ref, root_ref,
             bias_ref, ea_ref, za_ref, zb_ref, zc_ref, rt_ref, ea2_ref):
    ea3 = ea_ref[...].reshape(E // GRID // 8, 8, T)
    ea2_ref[...] = jnp.concatenate([ea3[:, j, :] for j in range(8)], axis=-1)
    xb = x_ref[...]
    mu = jnp.mean(xb, axis=-1, keepdims=True)
    var = jnp.mean((xb - mu) ** 2, axis=-1, keepdims=True)
    h = (xb - mu) * lax.rsqrt(var + 1e-5) * g_ref[...] + b_ref[...]
    h = jnp.maximum(h, 0.0)
    za_ref[...] = jnp.dot(h, m2a_ref[...], preferred_element_type=jnp.float32)
    zb_ref[...] = jnp.dot(h, m2b_ref[...], preferred_element_type=jnp.float32)
    zc_ref[...] = jnp.dot(h, m2c_ref[...], preferred_element_type=jnp.float32)
    rt_ref[...] = (
        jnp.dot(h, root_ref[...], preferred_element_type=jnp.float32)
        + bias_ref[...]
    )


def _tc_stage(x, ln_gamma, ln_beta, m2a, m2b, m2c, root, bias, ea):
    grid = (GRID,)
    return pl.pallas_call(
        _tc_body,
        grid=grid,
        in_specs=[
            pl.BlockSpec((ROWS, CIN), lambda i: (i, 0)),
            pl.BlockSpec((1, CIN), lambda i: (0, 0)),
            pl.BlockSpec((1, CIN), lambda i: (0, 0)),
            pl.BlockSpec((CIN, HALF), lambda i: (0, 0)),
            pl.BlockSpec((CIN, HALF), lambda i: (0, 0)),
            pl.BlockSpec((CIN, COUT), lambda i: (0, 0)),
            pl.BlockSpec((CIN, COUT), lambda i: (0, 0)),
            pl.BlockSpec((1, COUT), lambda i: (0, 0)),
            pl.BlockSpec((E // GRID, T), lambda i: (i, 0)),
        ],
        out_specs=[
            pl.BlockSpec((ROWS, HALF), lambda i: (i, 0)),
            pl.BlockSpec((ROWS, HALF), lambda i: (i, 0)),
            pl.BlockSpec((ROWS, COUT), lambda i: (i, 0)),
            pl.BlockSpec((ROWS, COUT), lambda i: (i, 0)),
            pl.BlockSpec((E // GRID // 8, 128), lambda i: (i, 0)),
        ],
        out_shape=[
            jax.ShapeDtypeStruct((NP, HALF), jnp.float32),
            jax.ShapeDtypeStruct((NP, HALF), jnp.float32),
            jax.ShapeDtypeStruct((NP, COUT), jnp.float32),
            jax.ShapeDtypeStruct((NP, COUT), jnp.float32),
            jax.ShapeDtypeStruct((E * T // 128, 128), jnp.float32),
        ],
    )(x, ln_gamma.reshape(1, CIN), ln_beta.reshape(1, CIN), m2a, m2b, m2c,
      root, bias.reshape(1, COUT), ea)


def _sc_body(za_hbm, zb_hbm, zc_hbm, eidx_hbm, ea_hbm, rt_hbm, out_hbm,
             aggr_sh, srcix, dstix, ea0, ea1, zra0, zra1, zrb0, zrb1,
             zrc0, zrc1, msg0, msg1, zbuf, sem0, sem1):
    cid = lax.axis_index("c")
    sid = lax.axis_index("s")
    wid = cid * NS + sid
    rpt = NP // NS

    def zrow(i, carry):
        zbuf[i, :] = jnp.zeros((COUT,), jnp.float32)
        return carry

    lax.fori_loop(0, ZROWS, zrow, 0, unroll=False)
    for m in range(rpt // ZROWS):
        pltpu.sync_copy(zbuf,
                        aggr_sh.at[pl.ds(sid * rpt + m * ZROWS, ZROWS)])

    @pl.when(cid == 0)
    def _():
        @pl.when(sid < NS - 1)
        def _():
            pltpu.sync_copy(rt_hbm.at[pl.ds(sid * rpt, rpt)],
                            aggr_sh.at[pl.ds(sid * rpt, rpt)])

        @pl.when(sid == NS - 1)
        def _():
            pltpu.sync_copy(rt_hbm.at[pl.ds((NS - 1) * rpt, N - (NS - 1) * rpt)],
                            aggr_sh.at[pl.ds((NS - 1) * rpt, N - (NS - 1) * rpt)])

    pltpu.sync_copy(eidx_hbm.at[0, wid], srcix)
    pltpu.sync_copy(eidx_hbm.at[1, wid], dstix)

    def ea_base(k):
        return jnp.minimum(wid * NCHUNK * EA_RPC + k * EA_RPC,
                           E * T // 128 - EA_RPC)

    def fetch(k, zra, zrb, zrc, ea, sem):
        idx = srcix.at[k]
        pltpu.async_copy(za_hbm.at[idx], zra, sem)
        pltpu.async_copy(zb_hbm.at[idx], zrb, sem)
        pltpu.async_copy(zc_hbm.at[idx], zrc, sem)
        pltpu.async_copy(ea_hbm.at[pl.ds(ea_base(k), EA_RPC)], ea, sem)

    def drain(k, zra, zrb, zrc, ea, sem):
        idx = srcix.at[k]
        pltpu.make_async_copy(za_hbm.at[idx], zra, sem).wait()
        pltpu.make_async_copy(zb_hbm.at[idx], zrb, sem).wait()
        pltpu.make_async_copy(zc_hbm.at[idx], zrc, sem).wait()
        pltpu.make_async_copy(ea_hbm.at[pl.ds(ea_base(k), EA_RPC)], ea,
                              sem).wait()

    def compute(k, zra, zrb, zrc, ea, msg):
        def erow(r, carry):
            for j in range(8):
                c = r * 8 + j
                ea_vec = ea[r, pl.ds(j * T, T)]
                acc = zrc[c, :]
                for t in range(8):
                    acc = acc + ea_vec[t] * zra[c, pl.ds(t * COUT, COUT)]
                for t in range(8):
                    acc = acc + ea_vec[8 + t] * zrb[c, pl.ds(t * COUT, COUT)]
                msg[c, :] = acc
            return carry

        lax.fori_loop(0, C // 8, erow, 0, unroll=False)
        pltpu.sync_copy(msg, aggr_sh.at[dstix.at[k]], add=True)

    fetch(0, zra0, zrb0, zrc0, ea0, sem0)
    plsc.subcore_barrier()
    npair = NCHUNK // 2

    def pair(j, carry):
        a = 2 * j
        fetch(a + 1, zra1, zrb1, zrc1, ea1, sem1)
        drain(a, zra0, zrb0, zrc0, ea0, sem0)
        compute(a, zra0, zrb0, zrc0, ea0, msg0)

        @pl.when(j < npair - 1)
        def _():
            fetch(a + 2, zra0, zrb0, zrc0, ea0, sem0)

        drain(a + 1, zra1, zrb1, zrc1, ea1, sem1)
        compute(a + 1, zra1, zrb1, zrc1, ea1, msg1)
        return carry

    lax.fori_loop(0, npair, pair, 0, unroll=False)

    plsc.subcore_barrier()

    pltpu.sync_copy(aggr_sh.at[pl.ds(sid * rpt, rpt)],
                    out_hbm.at[cid, pl.ds(sid * rpt, rpt)])


_sc_stage = pl.kernel(
    _sc_body,
    out_type=jax.ShapeDtypeStruct((NC, NP, COUT), jnp.float32),
    mesh=plsc.VectorSubcoreMesh(core_axis_name="c", subcore_axis_name="s"),
    compiler_params=pltpu.CompilerParams(use_tc_tiling_on_sc=False),
    scratch_types=[
        pltpu.VMEM_SHARED((NP, COUT), jnp.float32),
        pltpu.VMEM((NCHUNK, C), jnp.int32),
        pltpu.VMEM((NCHUNK, C), jnp.int32),
        pltpu.VMEM((EA_RPC, 128), jnp.float32),
        pltpu.VMEM((EA_RPC, 128), jnp.float32),
        pltpu.VMEM((C, HALF), jnp.float32),
        pltpu.VMEM((C, HALF), jnp.float32),
        pltpu.VMEM((C, HALF), jnp.float32),
        pltpu.VMEM((C, HALF), jnp.float32),
        pltpu.VMEM((C, COUT), jnp.float32),
        pltpu.VMEM((C, COUT), jnp.float32),
        pltpu.VMEM((C, COUT), jnp.float32),
        pltpu.VMEM((C, COUT), jnp.float32),
        pltpu.VMEM((ZROWS, COUT), jnp.float32),
        pltpu.SemaphoreType.DMA,
        pltpu.SemaphoreType.DMA,
    ],
)


def kernel(x, edge_index, edge_attr, ln_gamma, ln_beta, nn_W, nn_b, root, bias):
    m2 = nn_W.reshape(T, CIN, COUT).transpose(1, 0, 2).reshape(CIN, T * COUT)
    m2a = m2[:, :HALF]
    m2b = m2[:, HALF:]
    m2c = nn_b.reshape(CIN, COUT)

    za, zb, zc, rt, ea2 = _tc_stage(x, ln_gamma, ln_beta, m2a, m2b, m2c,
                                    root, bias, edge_attr)

    pad_n = EP - E
    pad_src = (jnp.arange(pad_n, dtype=jnp.int32) % N)[None]
    pad_dst = (N + jnp.arange(pad_n, dtype=jnp.int32) % (NP - N))[None]
    eidx = jnp.concatenate(
        [edge_index, jnp.concatenate([pad_src, pad_dst], axis=0)],
        axis=1).reshape(2, NW, NCHUNK, C)

    partial_sums = _sc_stage(za, zb, zc, eidx, ea2, rt)
    return (partial_sums[0] + partial_sums[1])[:N]


# speedup vs baseline: 1.1026x; 1.1026x over previous
import jax
import jax.numpy as jnp
from jax import lax
from jax.experimental import pallas as pl
from jax.experimental.pallas import tpu as pltpu
from jax.experimental.pallas import tpu_sc as plsc

N, E, CIN, COUT, T = 10000, 160000, 128, 16, 16
NP = 10240
HALF = T * COUT // 2

NC, NS = 2, 16
NW = NC * NS
C = 128
NCHUNK = 40
E_PER_W = NCHUNK * C
EP = NW * E_PER_W
ZROWS = 128

ROWS = 640
GRID = NP // ROWS


def _tc_body(x_ref, g_ref, b_ref, m2a_ref, m2b_ref, m2c_ref, root_ref,
             bias_ref, za_ref, zb_ref, zc_ref, rt_ref):
    xb = x_ref[...]
    mu = jnp.mean(xb, axis=-1, keepdims=True)
    var = jnp.mean((xb - mu) ** 2, axis=-1, keepdims=True)
    h = (xb - mu) * lax.rsqrt(var + 1e-5) * g_ref[...] + b_ref[...]
    h = jnp.maximum(h, 0.0)
    za_ref[...] = jnp.dot(h, m2a_ref[...], preferred_element_type=jnp.float32)
    zb_ref[...] = jnp.dot(h, m2b_ref[...], preferred_element_type=jnp.float32)
    zc_ref[...] = jnp.dot(h, m2c_ref[...], preferred_element_type=jnp.float32)
    rt_ref[...] = (
        jnp.dot(h, root_ref[...], preferred_element_type=jnp.float32)
        + bias_ref[...]
    )


def _tc_stage(x, ln_gamma, ln_beta, m2a, m2b, m2c, root, bias):
    grid = (GRID,)
    return pl.pallas_call(
        _tc_body,
        grid=grid,
        in_specs=[
            pl.BlockSpec((ROWS, CIN), lambda i: (i, 0)),
            pl.BlockSpec((1, CIN), lambda i: (0, 0)),
            pl.BlockSpec((1, CIN), lambda i: (0, 0)),
            pl.BlockSpec((CIN, HALF), lambda i: (0, 0)),
            pl.BlockSpec((CIN, HALF), lambda i: (0, 0)),
            pl.BlockSpec((CIN, COUT), lambda i: (0, 0)),
            pl.BlockSpec((CIN, COUT), lambda i: (0, 0)),
            pl.BlockSpec((1, COUT), lambda i: (0, 0)),
        ],
        out_specs=[
            pl.BlockSpec((ROWS, HALF), lambda i: (i, 0)),
            pl.BlockSpec((ROWS, HALF), lambda i: (i, 0)),
            pl.BlockSpec((ROWS, COUT), lambda i: (i, 0)),
            pl.BlockSpec((ROWS, COUT), lambda i: (i, 0)),
        ],
        out_shape=[
            jax.ShapeDtypeStruct((NP, HALF), jnp.float32),
            jax.ShapeDtypeStruct((NP, HALF), jnp.float32),
            jax.ShapeDtypeStruct((NP, COUT), jnp.float32),
            jax.ShapeDtypeStruct((NP, COUT), jnp.float32),
        ],
    )(x, ln_gamma.reshape(1, CIN), ln_beta.reshape(1, CIN), m2a, m2b, m2c,
      root, bias.reshape(1, COUT))


def _sc_body(za_hbm, zb_hbm, zc_hbm, eidx_hbm, ea_hbm, rt_hbm, out_hbm,
             aggr_sh, srcix, dstix, ea0, ea1, zra0, zra1, zrb0, zrb1,
             zrc0, zrc1, msg0, msg1, zbuf, sem0, sem1):
    cid = lax.axis_index("c")
    sid = lax.axis_index("s")
    wid = cid * NS + sid
    rpt = NP // NS

    def zrow(i, carry):
        zbuf[i, :] = jnp.zeros((COUT,), jnp.float32)
        return carry

    lax.fori_loop(0, ZROWS, zrow, 0, unroll=False)
    for m in range(rpt // ZROWS):
        pltpu.sync_copy(zbuf,
                        aggr_sh.at[pl.ds(sid * rpt + m * ZROWS, ZROWS)])

    @pl.when(cid == 0)
    def _():
        @pl.when(sid < NS - 1)
        def _():
            pltpu.sync_copy(rt_hbm.at[pl.ds(sid * rpt, rpt)],
                            aggr_sh.at[pl.ds(sid * rpt, rpt)])

        @pl.when(sid == NS - 1)
        def _():
            pltpu.sync_copy(rt_hbm.at[pl.ds((NS - 1) * rpt, N - (NS - 1) * rpt)],
                            aggr_sh.at[pl.ds((NS - 1) * rpt, N - (NS - 1) * rpt)])

    pltpu.sync_copy(eidx_hbm.at[0, wid], srcix)
    pltpu.sync_copy(eidx_hbm.at[1, wid], dstix)

    def ea_base(k):
        return jnp.minimum(wid * E_PER_W + k * C, E - C)

    def fetch(k, zra, zrb, zrc, ea, sem):
        idx = srcix.at[k]
        pltpu.async_copy(za_hbm.at[idx], zra, sem)
        pltpu.async_copy(zb_hbm.at[idx], zrb, sem)
        pltpu.async_copy(zc_hbm.at[idx], zrc, sem)
        pltpu.async_copy(ea_hbm.at[pl.ds(ea_base(k), C)], ea, sem)

    def drain(k, zra, zrb, zrc, ea, sem):
        idx = srcix.at[k]
        pltpu.make_async_copy(za_hbm.at[idx], zra, sem).wait()
        pltpu.make_async_copy(zb_hbm.at[idx], zrb, sem).wait()
        pltpu.make_async_copy(zc_hbm.at[idx], zrc, sem).wait()
        pltpu.make_async_copy(ea_hbm.at[pl.ds(ea_base(k), C)], ea, sem).wait()

    def compute(k, zra, zrb, zrc, ea, msg):
        def edge(c, carry):
            ea_vec = ea[c, :]
            acc = zrc[c, :]
            for t in range(8):
                acc = acc + ea_vec[t] * zra[c, pl.ds(t * COUT, COUT)]
            for t in range(8):
                acc = acc + ea_vec[8 + t] * zrb[c, pl.ds(t * COUT, COUT)]
            msg[c, :] = acc
            return carry

        lax.fori_loop(0, C, edge, 0, unroll=False)
        pltpu.sync_copy(msg, aggr_sh.at[dstix.at[k]], add=True)

    fetch(0, zra0, zrb0, zrc0, ea0, sem0)
    plsc.subcore_barrier()
    npair = NCHUNK // 2

    def pair(j, carry):
        a = 2 * j
        fetch(a + 1, zra1, zrb1, zrc1, ea1, sem1)
        drain(a, zra0, zrb0, zrc0, ea0, sem0)
        compute(a, zra0, zrb0, zrc0, ea0, msg0)

        @pl.when(j < npair - 1)
        def _():
            fetch(a + 2, zra0, zrb0, zrc0, ea0, sem0)

        drain(a + 1, zra1, zrb1, zrc1, ea1, sem1)
        compute(a + 1, zra1, zrb1, zrc1, ea1, msg1)
        return carry

    lax.fori_loop(0, npair, pair, 0, unroll=False)

    plsc.subcore_barrier()

    pltpu.sync_copy(aggr_sh.at[pl.ds(sid * rpt, rpt)],
                    out_hbm.at[cid, pl.ds(sid * rpt, rpt)])


_sc_stage = pl.kernel(
    _sc_body,
    out_type=jax.ShapeDtypeStruct((NC, NP, COUT), jnp.float32),
    mesh=plsc.VectorSubcoreMesh(core_axis_name="c", subcore_axis_name="s"),
    compiler_params=pltpu.CompilerParams(use_tc_tiling_on_sc=False),
    scratch_types=[
        pltpu.VMEM_SHARED((NP, COUT), jnp.float32),
        pltpu.VMEM((NCHUNK, C), jnp.int32),
        pltpu.VMEM((NCHUNK, C), jnp.int32),
        pltpu.VMEM((C, T), jnp.float32),
        pltpu.VMEM((C, T), jnp.float32),
        pltpu.VMEM((C, HALF), jnp.float32),
        pltpu.VMEM((C, HALF), jnp.float32),
        pltpu.VMEM((C, HALF), jnp.float32),
        pltpu.VMEM((C, HALF), jnp.float32),
        pltpu.VMEM((C, COUT), jnp.float32),
        pltpu.VMEM((C, COUT), jnp.float32),
        pltpu.VMEM((C, COUT), jnp.float32),
        pltpu.VMEM((C, COUT), jnp.float32),
        pltpu.VMEM((ZROWS, COUT), jnp.float32),
        pltpu.SemaphoreType.DMA,
        pltpu.SemaphoreType.DMA,
    ],
)


def kernel(x, edge_index, edge_attr, ln_gamma, ln_beta, nn_W, nn_b, root, bias):
    m2 = nn_W.reshape(T, CIN, COUT).transpose(1, 0, 2).reshape(CIN, T * COUT)
    m2a = m2[:, :HALF]
    m2b = m2[:, HALF:]
    m2c = nn_b.reshape(CIN, COUT)

    za, zb, zc, rt = _tc_stage(x, ln_gamma, ln_beta, m2a, m2b, m2c, root, bias)

    pad_n = EP - E
    pad_src = (jnp.arange(pad_n, dtype=jnp.int32) % N)[None]
    pad_dst = (N + jnp.arange(pad_n, dtype=jnp.int32) % (NP - N))[None]
    eidx = jnp.concatenate(
        [edge_index, jnp.concatenate([pad_src, pad_dst], axis=0)],
        axis=1).reshape(2, NW, NCHUNK, C)

    partial_sums = _sc_stage(za, zb, zc, eidx, edge_attr, rt)
    return (partial_sums[0] + partial_sums[1])[:N]
